# baseline (device time: 84628 ns/iter reference)
import jax
import jax.numpy as jnp
from jax import lax
from jax.experimental import pallas as pl
from jax.experimental.pallas import tpu as pltpu

N_DEV = 16
B_PER = 2
SQ = 128
D = 512
H_PER = 4
DH = 64
T = B_PER * SQ
HD = H_PER * DH

NS = 4
ROWS = T // NS

R_HOPS = N_DEV // 2
L_HOPS = N_DEV - 1 - R_HOPS

F32 = jnp.float32
BF16 = jnp.bfloat16


def _body(x_ref, wq_ref, wk_ref, wv_ref, wo_ref, cos_ref, sina_ref, sinb_ref,
          out_ref, xg_ref, p_ref,
          ag_sr, ag_rr, ag_sl, ag_rl,
          sbr_ref, sbl_ref, cr_ref, cl_ref,
          rs_sr, rs_rr, rs_sl, rs_rl):
    me = lax.axis_index("i")
    left = lax.rem(me + N_DEV - 1, N_DEV)
    right = lax.rem(me + 1, N_DEV)

    barrier_sem = pltpu.get_barrier_semaphore()
    for nbr in (left, right):
        pl.semaphore_signal(
            barrier_sem, inc=1,
            device_id=(nbr,), device_id_type=pl.DeviceIdType.MESH,
        )
    pl.semaphore_wait(barrier_sem, 2)

    wq = wq_ref[...]
    wk = wk_ref[...]
    wv = wv_ref[...]
    wo = wo_ref[...]
    cosm = cos_ref[...]
    sina = sina_ref[...]
    sinb = sinb_ref[...]

    def rope(t):
        tm1 = jnp.concatenate([t[:, 1:], t[:, :1]], axis=1)
        tp1 = jnp.concatenate([t[:, -1:], t[:, :-1]], axis=1)
        return (t * cosm + tm1 * sina + tp1 * sinb).astype(BF16)

    def compute_chunk(c, xc):
        q = rope(jnp.dot(xc, wq, preferred_element_type=F32))
        k = rope(jnp.dot(xc, wk, preferred_element_type=F32))
        v = jnp.dot(xc, wv, preferred_element_type=F32).astype(BF16)

        s_list = []
        for b in range(B_PER):
            for h in range(H_PER):
                qh = q[b * SQ:(b + 1) * SQ, h * DH:(h + 1) * DH]
                kh = k[b * SQ:(b + 1) * SQ, h * DH:(h + 1) * DH]
                s_list.append(lax.dot_general(
                    qh, kh, (((1,), (1,)), ((), ())),
                    preferred_element_type=F32,
                ))
        s = jnp.concatenate(s_list, axis=0) * 0.125
        e = jnp.exp(s)
        denom = jnp.sum(e, axis=-1, keepdims=True)
        eb = e.astype(BF16)

        ctx_rows = []
        for b in range(B_PER):
            ctx_heads = []
            for h in range(H_PER):
                i = b * H_PER + h
                ebh = eb[i * SQ:(i + 1) * SQ, :]
                vh = v[b * SQ:(b + 1) * SQ, h * DH:(h + 1) * DH]
                raw = jnp.dot(ebh, vh, preferred_element_type=F32)
                ctx_heads.append(
                    (raw / denom[i * SQ:(i + 1) * SQ, :]).astype(BF16))
            ctx_rows.append(jnp.concatenate(ctx_heads, axis=1))
        ctx = jnp.concatenate(ctx_rows, axis=0)
        partial = jnp.dot(ctx, wo, preferred_element_type=F32)
        for j in range(NS):
            p_ref[c * NS + j] = partial[j * ROWS:(j + 1) * ROWS, :]

    def gathered(c):
        return jnp.concatenate(
            [xg_ref[c * NS + j] for j in range(NS)], axis=0)

    def ag_desc(c, h, j, to_right):
        src = x_ref.at[j] if h == 0 else xg_ref.at[c * NS + j]
        return pltpu.make_async_remote_copy(
            src_ref=src,
            dst_ref=xg_ref.at[c * NS + j],
            send_sem=(ag_sr if to_right else ag_sl).at[h * NS + j],
            recv_sem=(ag_rr if to_right else ag_rl).at[h * NS + j],
            device_id=(right if to_right else left,),
            device_id_type=pl.DeviceIdType.MESH,
        )

    r_descs = [[None] * NS for _ in range(R_HOPS)]
    l_descs = [[None] * NS for _ in range(L_HOPS)]
    for j in range(NS):
        r_descs[0][j] = ag_desc(me, 0, j, True)
        r_descs[0][j].start()
        l_descs[0][j] = ag_desc(me, 0, j, False)
        l_descs[0][j].start()

    for h in range(1, R_HOPS):
        c_r = lax.rem(me - h + N_DEV, N_DEV)
        c_l = lax.rem(me + h, N_DEV)
        for j in range(NS):
            r_descs[h - 1][j].wait_recv()
            r_descs[h][j] = ag_desc(c_r, h, j, True)
            r_descs[h][j].start()
        if h < L_HOPS:
            for j in range(NS):
                l_descs[h - 1][j].wait_recv()
                l_descs[h][j] = ag_desc(c_l, h, j, False)
                l_descs[h][j].start()
        else:
            for j in range(NS):
                l_descs[L_HOPS - 1][j].wait_recv()
        compute_chunk(c_r, gathered(c_r))

    rsr = [[None] * NS for _ in range(R_HOPS)]
    rsl = [[None] * NS for _ in range(L_HOPS)]

    def rs_send(s, j, c, to_right):
        sb_ref = sbr_ref if to_right else sbl_ref
        cm_ref = cr_ref if to_right else cl_ref
        if s == 0:
            sb_ref[(s % 2) * NS + j] = p_ref[c * NS + j].astype(BF16)
        else:
            (rsr if to_right else rsl)[s - 1][j].wait_recv()
            sb_ref[(s % 2) * NS + j] = (
                p_ref[c * NS + j]
                + cm_ref[(s - 1) * NS + j].astype(F32)
            ).astype(BF16)
        d = pltpu.make_async_remote_copy(
            src_ref=sb_ref.at[(s % 2) * NS + j],
            dst_ref=cm_ref.at[s * NS + j],
            send_sem=(rs_sr if to_right else rs_sl).at[s * NS + j],
            recv_sem=(rs_rr if to_right else rs_rl).at[s * NS + j],
            device_id=(right if to_right else left,),
            device_id_type=pl.DeviceIdType.MESH,
        )
        d.start()
        (rsr if to_right else rsl)[s][j] = d

    c_l0 = lax.rem(me - L_HOPS + N_DEV, N_DEV)
    for j in range(NS):
        rs_send(0, j, c_l0, False)
    for j in range(NS):
        r_descs[R_HOPS - 1][j].wait_recv()
    c_far = lax.rem(me + R_HOPS, N_DEV)
    compute_chunk(c_far, gathered(c_far))
    for j in range(NS):
        rs_send(0, j, c_far, True)

    for s in range(1, R_HOPS):
        c_r = lax.rem(me + R_HOPS - s, N_DEV)
        compute_chunk(c_r, gathered(c_r))
        for j in range(NS):
            if s >= 2:
                rsr[s - 2][j].wait_send()
            rs_send(s, j, c_r, True)
        if s < L_HOPS:
            c_l = lax.rem(me - L_HOPS + s + N_DEV, N_DEV)
            for j in range(NS):
                if s >= 2:
                    rsl[s - 2][j].wait_send()
                rs_send(s, j, c_l, False)

    compute_chunk(me, x_ref[...].reshape(T, D))
    for j in range(NS):
        rsr[R_HOPS - 1][j].wait_recv()
        rsl[L_HOPS - 1][j].wait_recv()

    qpb = NS // B_PER
    for j in range(NS):
        b, r0 = j // qpb, (j % qpb) * ROWS
        out_ref[b, r0:r0 + ROWS, :] = (
            p_ref[me * NS + j]
            + cr_ref[(R_HOPS - 1) * NS + j].astype(F32)
            + cl_ref[(L_HOPS - 1) * NS + j].astype(F32)
        )

    for descs in (r_descs, l_descs):
        for group in descs:
            for d in group:
                d.wait_send()
    for s in (R_HOPS - 2, R_HOPS - 1):
        for j in range(NS):
            rsr[s][j].wait_send()
    for s in (L_HOPS - 2, L_HOPS - 1):
        for j in range(NS):
            rsl[s][j].wait_send()


def kernel(x, Wq, Wk, Wv, Wo):
    inv = 1.0 / (10000.0 ** (jnp.arange(0, DH, 2, dtype=F32) / DH))
    pos = jnp.arange(SQ, dtype=F32)[:, None] * inv[None, :]
    cos = jnp.repeat(jnp.cos(pos), 2, axis=-1)
    sin = jnp.repeat(jnp.sin(pos), 2, axis=-1)
    cosm = jnp.tile(cos, (B_PER, H_PER))
    sinm = jnp.tile(sin, (B_PER, H_PER))
    even = (jnp.arange(HD) % 2 == 0)[None, :]
    sina = jnp.where(even, -sinm, 0.0)
    sinb = jnp.where(even, 0.0, sinm)

    return pl.pallas_call(
        _body,
        out_shape=jax.ShapeDtypeStruct((B_PER, SQ, D), F32),
        in_specs=[pl.BlockSpec(memory_space=pltpu.VMEM)] * 8,
        out_specs=pl.BlockSpec(memory_space=pltpu.VMEM),
        scratch_shapes=[
            pltpu.VMEM((N_DEV * NS, ROWS, D), BF16),
            pltpu.VMEM((N_DEV * NS, ROWS, D), F32),
            pltpu.SemaphoreType.DMA((R_HOPS * NS,)),
            pltpu.SemaphoreType.DMA((R_HOPS * NS,)),
            pltpu.SemaphoreType.DMA((L_HOPS * NS,)),
            pltpu.SemaphoreType.DMA((L_HOPS * NS,)),
            pltpu.VMEM((2 * NS, ROWS, D), BF16),
            pltpu.VMEM((2 * NS, ROWS, D), BF16),
            pltpu.VMEM((R_HOPS * NS, ROWS, D), BF16),
            pltpu.VMEM((L_HOPS * NS, ROWS, D), BF16),
            pltpu.SemaphoreType.DMA((R_HOPS * NS,)),
            pltpu.SemaphoreType.DMA((R_HOPS * NS,)),
            pltpu.SemaphoreType.DMA((L_HOPS * NS,)),
            pltpu.SemaphoreType.DMA((L_HOPS * NS,)),
        ],
        compiler_params=pltpu.CompilerParams(collective_id=0),
    )(
        x.reshape(NS, ROWS, D).astype(BF16),
        Wq.astype(BF16), Wk.astype(BF16), Wv.astype(BF16), Wo.astype(BF16),
        cosm, sina, sinb,
    )


# device time: 74182 ns/iter; 1.1408x vs baseline; 1.1408x over previous
import jax
import jax.numpy as jnp
import numpy as np
from jax import lax
from jax.experimental import pallas as pl
from jax.experimental.pallas import tpu as pltpu

N_DEV = 16

_PERM = np.array([0, 1, 5, 9, 13, 14, 10, 6, 2, 3, 7, 11, 15, 12, 8, 4],
                 dtype=np.int32)
_INV = np.empty_like(_PERM)
_INV[_PERM] = np.arange(N_DEV, dtype=np.int32)
_RIGHT_OF = _PERM[(_INV + 1) % N_DEV]
_LEFT_OF = _PERM[(_INV - 1) % N_DEV]
B_PER = 2
SQ = 128
D = 512
H_PER = 4
DH = 64
T = B_PER * SQ
HD = H_PER * DH

R_HOPS = N_DEV // 2
L_HOPS = N_DEV - 1 - R_HOPS

F32 = jnp.float32
BF16 = jnp.bfloat16


def _body(pos_ref, ldev_ref, rdev_ref,
          x_ref, wq_ref, wk_ref, wv_ref, wo_ref, cos_ref, sina_ref, sinb_ref,
          out_ref, xg_ref, p_ref,
          ag_sr, ag_rr, ag_sl, ag_rl,
          sbr_ref, sbl_ref, cr_ref, cl_ref,
          rs_sr, rs_rr, rs_sl, rs_rl):
    me = pos_ref[0]
    left = ldev_ref[0]
    right = rdev_ref[0]

    barrier_sem = pltpu.get_barrier_semaphore()
    for nbr in (left, right):
        pl.semaphore_signal(
            barrier_sem, inc=1,
            device_id=(nbr,), device_id_type=pl.DeviceIdType.MESH,
        )
    pl.semaphore_wait(barrier_sem, 2)

    wq = wq_ref[...]
    wk = wk_ref[...]
    wv = wv_ref[...]
    wo = wo_ref[...]
    cosm = cos_ref[...]
    sina = sina_ref[...]
    sinb = sinb_ref[...]

    def rope(t):
        tm1 = jnp.concatenate([t[:, 1:], t[:, :1]], axis=1)
        tp1 = jnp.concatenate([t[:, -1:], t[:, :-1]], axis=1)
        return (t * cosm + tm1 * sina + tp1 * sinb).astype(BF16)

    def compute_chunk(c, xc):
        q = rope(jnp.dot(xc, wq, preferred_element_type=F32))
        k = rope(jnp.dot(xc, wk, preferred_element_type=F32))
        v = jnp.dot(xc, wv, preferred_element_type=F32).astype(BF16)

        s_list = []
        for b in range(B_PER):
            for h in range(H_PER):
                qh = q[b * SQ:(b + 1) * SQ, h * DH:(h + 1) * DH]
                kh = k[b * SQ:(b + 1) * SQ, h * DH:(h + 1) * DH]
                s_list.append(lax.dot_general(
                    qh, kh, (((1,), (1,)), ((), ())),
                    preferred_element_type=F32,
                ))
        s = jnp.concatenate(s_list, axis=0) * 0.125
        e = jnp.exp(s)
        denom = jnp.sum(e, axis=-1, keepdims=True)
        eb = e.astype(BF16)

        ctx_rows = []
        for b in range(B_PER):
            ctx_heads = []
            for h in range(H_PER):
                i = b * H_PER + h
                ebh = eb[i * SQ:(i + 1) * SQ, :]
                vh = v[b * SQ:(b + 1) * SQ, h * DH:(h + 1) * DH]
                raw = jnp.dot(ebh, vh, preferred_element_type=F32)
                ctx_heads.append(
                    (raw / denom[i * SQ:(i + 1) * SQ, :]).astype(BF16))
            ctx_rows.append(jnp.concatenate(ctx_heads, axis=1))
        ctx = jnp.concatenate(ctx_rows, axis=0)
        partial = jnp.dot(ctx, wo, preferred_element_type=F32)
        p_ref[c * 2] = partial[:SQ, :]
        p_ref[c * 2 + 1] = partial[SQ:, :]

    def gathered(c):
        return jnp.concatenate([xg_ref[c * 2], xg_ref[c * 2 + 1]], axis=0)

    def ag_desc(c, h, j, to_right):
        src = x_ref.at[j] if h == 0 else xg_ref.at[c * 2 + j]
        return pltpu.make_async_remote_copy(
            src_ref=src,
            dst_ref=xg_ref.at[c * 2 + j],
            send_sem=(ag_sr if to_right else ag_sl).at[h * 2 + j],
            recv_sem=(ag_rr if to_right else ag_rl).at[h * 2 + j],
            device_id=(right if to_right else left,),
            device_id_type=pl.DeviceIdType.MESH,
        )

    r_descs = [[None, None] for _ in range(R_HOPS)]
    l_descs = [[None, None] for _ in range(L_HOPS)]
    for j in range(2):
        r_descs[0][j] = ag_desc(me, 0, j, True)
        r_descs[0][j].start()
        l_descs[0][j] = ag_desc(me, 0, j, False)
        l_descs[0][j].start()

    for h in range(1, R_HOPS):
        c_r = lax.rem(me - h + N_DEV, N_DEV)
        c_l = lax.rem(me + h, N_DEV)
        for j in range(2):
            r_descs[h - 1][j].wait_recv()
            r_descs[h][j] = ag_desc(c_r, h, j, True)
            r_descs[h][j].start()
        if h < L_HOPS:
            for j in range(2):
                l_descs[h - 1][j].wait_recv()
                l_descs[h][j] = ag_desc(c_l, h, j, False)
                l_descs[h][j].start()
        else:
            for j in range(2):
                l_descs[L_HOPS - 1][j].wait_recv()
        compute_chunk(c_r, gathered(c_r))

    for j in range(2):
        r_descs[R_HOPS - 1][j].wait_recv()

    for descs in (r_descs, l_descs):
        for pair in descs:
            for d in pair:
                d.wait_send()

    rsr = [[None, None] for _ in range(R_HOPS)]
    rsl = [[None, None] for _ in range(L_HOPS)]

    def rs_send(s, j, c, to_right):
        sb_ref = sbr_ref if to_right else sbl_ref
        cm_ref = cr_ref if to_right else cl_ref
        if s == 0:
            sb_ref[(s % 2) * 2 + j] = p_ref[c * 2 + j].astype(BF16)
        else:
            (rsr if to_right else rsl)[s - 1][j].wait_recv()
            sb_ref[(s % 2) * 2 + j] = (
                p_ref[c * 2 + j]
                + cm_ref[(s - 1) * 2 + j].astype(F32)
            ).astype(BF16)
        d = pltpu.make_async_remote_copy(
            src_ref=sb_ref.at[(s % 2) * 2 + j],
            dst_ref=cm_ref.at[s * 2 + j],
            send_sem=(rs_sr if to_right else rs_sl).at[s * 2 + j],
            recv_sem=(rs_rr if to_right else rs_rl).at[s * 2 + j],
            device_id=(right if to_right else left,),
            device_id_type=pl.DeviceIdType.MESH,
        )
        d.start()
        (rsr if to_right else rsl)[s][j] = d

    c_l0 = lax.rem(me - L_HOPS + N_DEV, N_DEV)
    for j in range(2):
        rs_send(0, j, c_l0, False)
    c_far = lax.rem(me + R_HOPS, N_DEV)
    compute_chunk(c_far, gathered(c_far))
    for j in range(2):
        rs_send(0, j, c_far, True)

    for s in range(1, R_HOPS):
        c_r = lax.rem(me + R_HOPS - s, N_DEV)
        compute_chunk(c_r, gathered(c_r))
        for j in range(2):
            if s >= 2:
                rsr[s - 2][j].wait_send()
            rs_send(s, j, c_r, True)
        if s < L_HOPS:
            c_l = lax.rem(me - L_HOPS + s + N_DEV, N_DEV)
            for j in range(2):
                if s >= 2:
                    rsl[s - 2][j].wait_send()
                rs_send(s, j, c_l, False)

    compute_chunk(me, x_ref[...].reshape(T, D))
    for j in range(2):
        rsr[R_HOPS - 1][j].wait_recv()
        rsl[L_HOPS - 1][j].wait_recv()
    for j in range(2):
        out_ref[j] = (
            p_ref[me * 2 + j]
            + cr_ref[(R_HOPS - 1) * 2 + j].astype(F32)
            + cl_ref[(L_HOPS - 1) * 2 + j].astype(F32)
        )
    for s in (R_HOPS - 2, R_HOPS - 1):
        for j in range(2):
            rsr[s][j].wait_send()
    for s in (L_HOPS - 2, L_HOPS - 1):
        for j in range(2):
            rsl[s][j].wait_send()


def kernel(x, Wq, Wk, Wv, Wo):
    inv = 1.0 / (10000.0 ** (jnp.arange(0, DH, 2, dtype=F32) / DH))
    pos = jnp.arange(SQ, dtype=F32)[:, None] * inv[None, :]
    cos = jnp.repeat(jnp.cos(pos), 2, axis=-1)
    sin = jnp.repeat(jnp.sin(pos), 2, axis=-1)
    cosm = jnp.tile(cos, (B_PER, H_PER))
    sinm = jnp.tile(sin, (B_PER, H_PER))
    even = (jnp.arange(HD) % 2 == 0)[None, :]
    sina = jnp.where(even, -sinm, 0.0)
    sinb = jnp.where(even, 0.0, sinm)

    me_l = lax.axis_index("i")
    pos = jnp.asarray(_INV)[me_l].reshape(1)
    ldev = jnp.asarray(_LEFT_OF)[me_l].reshape(1)
    rdev = jnp.asarray(_RIGHT_OF)[me_l].reshape(1)

    return pl.pallas_call(
        _body,
        out_shape=jax.ShapeDtypeStruct((B_PER, SQ, D), F32),
        in_specs=[pl.BlockSpec(memory_space=pltpu.SMEM)] * 3
        + [pl.BlockSpec(memory_space=pltpu.VMEM)] * 8,
        out_specs=pl.BlockSpec(memory_space=pltpu.VMEM),
        scratch_shapes=[
            pltpu.VMEM((N_DEV * 2, SQ, D), BF16),
            pltpu.VMEM((N_DEV * 2, SQ, D), F32),
            pltpu.SemaphoreType.DMA((R_HOPS * 2,)),
            pltpu.SemaphoreType.DMA((R_HOPS * 2,)),
            pltpu.SemaphoreType.DMA((L_HOPS * 2,)),
            pltpu.SemaphoreType.DMA((L_HOPS * 2,)),
            pltpu.VMEM((4, SQ, D), BF16),
            pltpu.VMEM((4, SQ, D), BF16),
            pltpu.VMEM((R_HOPS * 2, SQ, D), BF16),
            pltpu.VMEM((L_HOPS * 2, SQ, D), BF16),
            pltpu.SemaphoreType.DMA((R_HOPS * 2,)),
            pltpu.SemaphoreType.DMA((R_HOPS * 2,)),
            pltpu.SemaphoreType.DMA((L_HOPS * 2,)),
            pltpu.SemaphoreType.DMA((L_HOPS * 2,)),
        ],
        compiler_params=pltpu.CompilerParams(collective_id=0),
    )(
        pos, ldev, rdev,
        x.astype(BF16),
        Wq.astype(BF16), Wk.astype(BF16), Wv.astype(BF16), Wo.astype(BF16),
        cosm, sina, sinb,
    )
